# Initial kernel scaffold; baseline (speedup 1.0000x reference)
import functools

import jax
import jax.numpy as jnp
from jax import lax
from jax.experimental import pallas as pl
from jax.experimental.pallas import tpu as pltpu
from jax.experimental.pallas import tpu_sc as plsc

N_NODES = 10000
D_FEAT = 128
N_EDGES = 320000

NC = 2
NS = 16
NW = NC * NS
L = 16

E_W = N_EDGES // NW
CHUNK = 80
N_CHUNK = E_W // CHUNK
ROWS_TILE = N_NODES // NS

_mesh = plsc.VectorSubcoreMesh(core_axis_name="c", subcore_axis_name="s")


@functools.partial(
    pl.kernel,
    out_type=jax.ShapeDtypeStruct((NC, N_NODES, D_FEAT), jnp.float32),
    mesh=_mesh,
    scratch_types=[
        pltpu.VMEM((CHUNK,), jnp.int32),
        pltpu.VMEM((CHUNK,), jnp.int32),
        pltpu.VMEM((CHUNK, D_FEAT), jnp.float32),
        pltpu.MemorySpace.VMEM_SHARED((N_NODES, D_FEAT), jnp.float32),
        pltpu.SemaphoreType.DMA,
    ],
)
def _segsum(x_hbm, src_hbm, dst_hbm, zeros_hbm, part_hbm,
            src_v, dst_v, rows_v, acc_sh, sem):
    cid = lax.axis_index("c")
    sid = lax.axis_index("s")
    wid = cid * NS + sid
    r0 = sid * ROWS_TILE
    pltpu.sync_copy(zeros_hbm.at[pl.ds(r0, ROWS_TILE)],
                    acc_sh.at[pl.ds(r0, ROWS_TILE)])
    plsc.subcore_barrier()

    base_w = wid * E_W

    def body(i, carry):
        base = base_w + i * CHUNK
        pltpu.sync_copy(src_hbm.at[pl.ds(base, CHUNK)], src_v)
        pltpu.sync_copy(dst_hbm.at[pl.ds(base, CHUNK)], dst_v)
        pltpu.async_copy(x_hbm.at[src_v], rows_v, sem).wait()
        pltpu.sync_copy(rows_v, acc_sh.at[dst_v], add=True)
        return carry

    lax.fori_loop(0, N_CHUNK, body, 0)
    plsc.subcore_barrier()
    pltpu.sync_copy(acc_sh.at[pl.ds(r0, ROWS_TILE)],
                    part_hbm.at[cid, pl.ds(r0, ROWS_TILE)])


def _mm_body(p_ref, w_ref, b_ref, z_ref):
    agg = p_ref[0] + p_ref[1]
    z = jnp.dot(agg, w_ref[...], preferred_element_type=jnp.float32)
    z_ref[...] = jnp.maximum(z + b_ref[...], 0.0)


def _encode_mm(parts, W, b2d):
    blk = 1000
    return pl.pallas_call(
        _mm_body,
        grid=(N_NODES // blk,),
        in_specs=[
            pl.BlockSpec((NC, blk, D_FEAT), lambda i: (0, i, 0)),
            pl.BlockSpec((D_FEAT, D_FEAT), lambda i: (0, 0)),
            pl.BlockSpec((1, D_FEAT), lambda i: (0, 0)),
        ],
        out_specs=pl.BlockSpec((blk, D_FEAT), lambda i: (i, 0)),
        out_shape=jax.ShapeDtypeStruct((N_NODES, D_FEAT), jnp.float32),
    )(parts, W, b2d)


@functools.partial(
    pl.kernel,
    out_type=jax.ShapeDtypeStruct((N_EDGES,), jnp.float32),
    mesh=_mesh,
    scratch_types=[
        pltpu.VMEM((CHUNK,), jnp.int32),
        pltpu.VMEM((CHUNK,), jnp.int32),
        pltpu.VMEM((CHUNK, D_FEAT), jnp.float32),
        pltpu.VMEM((CHUNK, D_FEAT), jnp.float32),
        pltpu.VMEM((CHUNK,), jnp.float32),
        pltpu.SemaphoreType.DMA,
        pltpu.SemaphoreType.DMA,
    ],
)
def _decode(z_hbm, src_hbm, dst_hbm, out_hbm,
            src_v, dst_v, srows_v, drows_v, out_v, sem_s, sem_d):
    cid = lax.axis_index("c")
    sid = lax.axis_index("s")
    wid = cid * NS + sid
    base_w = wid * E_W
    lane = lax.iota(jnp.int32, L)

    def body(i, carry):
        base = base_w + i * CHUNK
        pltpu.sync_copy(src_hbm.at[pl.ds(base, CHUNK)], src_v)
        pltpu.sync_copy(dst_hbm.at[pl.ds(base, CHUNK)], dst_v)
        cps = pltpu.async_copy(z_hbm.at[src_v], srows_v, sem_s)
        cpd = pltpu.async_copy(z_hbm.at[dst_v], drows_v, sem_d)
        cps.wait()
        cpd.wait()

        def group(g, carry):
            res = jnp.zeros((L,), jnp.float32)
            for j in range(L):
                acc = jnp.zeros((L,), jnp.float32)
                e = g * L + j
                for c in range(D_FEAT // L):
                    sv = srows_v[e, pl.ds(c * L, L)]
                    dv = drows_v[e, pl.ds(c * L, L)]
                    acc = acc + sv * dv
                tot = lax.reduce_sum_p.bind(acc, axes=(0,))
                res = jnp.where(lane == j, tot, res)
            out_v[pl.ds(g * L, L)] = res
            return carry

        lax.fori_loop(0, CHUNK // L, group, 0)
        pltpu.sync_copy(out_v, out_hbm.at[pl.ds(base, CHUNK)])
        return carry

    lax.fori_loop(0, N_CHUNK, body, 0)


def kernel(x, edge_index, W, b):
    src = edge_index[0].astype(jnp.int32)
    dst = edge_index[1].astype(jnp.int32)
    zeros = jnp.zeros((N_NODES, D_FEAT), jnp.float32)
    parts = _segsum(x, src, dst, zeros)
    z = _encode_mm(parts, W, b.reshape(1, D_FEAT))
    return _decode(z, src, dst)


# speedup vs baseline: 3.9341x; 3.9341x over previous
import functools

import jax
import jax.numpy as jnp
from jax import lax
from jax.experimental import pallas as pl
from jax.experimental.pallas import tpu as pltpu
from jax.experimental.pallas import tpu_sc as plsc

N_NODES = 10000
D_FEAT = 128
N_EDGES = 320000

NC = 2
NS = 16
NW = NC * NS
L = 16

E_W = N_EDGES // NW
CHUNK = 80
N_CHUNK = E_W // CHUNK
N_PAD = 10112
ROWS_TILE = N_PAD // NS

_mesh = plsc.VectorSubcoreMesh(core_axis_name="c", subcore_axis_name="s")

_GDN = lax.GatherDimensionNumbers(
    offset_dims=(), collapsed_slice_dims=(0,), start_index_map=(0,))


def _lane_shuffle(v, perm):
    return lax.gather(v, perm[:, None], _GDN, slice_sizes=(1,),
                      mode=lax.GatherScatterMode.PROMISE_IN_BOUNDS)


@functools.partial(
    pl.kernel,
    out_type=jax.ShapeDtypeStruct((NC, N_PAD, D_FEAT), jnp.float32),
    mesh=_mesh,
    scratch_types=[
        pltpu.VMEM((CHUNK,), jnp.int32),
        pltpu.VMEM((CHUNK,), jnp.int32),
        pltpu.VMEM((CHUNK, D_FEAT), jnp.float32),
        pltpu.MemorySpace.VMEM_SHARED((N_PAD, D_FEAT), jnp.float32),
        pltpu.SemaphoreType.DMA,
    ],
)
def _segsum(x_hbm, src_hbm, dst_hbm, zeros_hbm, part_hbm,
            src_v, dst_v, rows_v, acc_sh, sem):
    cid = lax.axis_index("c")
    sid = lax.axis_index("s")
    wid = cid * NS + sid
    r0 = sid * ROWS_TILE
    pltpu.sync_copy(zeros_hbm.at[pl.ds(r0, ROWS_TILE)],
                    acc_sh.at[pl.ds(r0, ROWS_TILE)])
    plsc.subcore_barrier()

    base_w = wid * E_W

    def body(i, carry):
        base = base_w + i * CHUNK
        pltpu.sync_copy(src_hbm.at[pl.ds(base, CHUNK)], src_v)
        pltpu.sync_copy(dst_hbm.at[pl.ds(base, CHUNK)], dst_v)
        pltpu.async_copy(x_hbm.at[src_v], rows_v, sem).wait()
        pltpu.sync_copy(rows_v, acc_sh.at[dst_v], add=True)
        return carry

    lax.fori_loop(0, N_CHUNK, body, 0)
    plsc.subcore_barrier()
    pltpu.sync_copy(acc_sh.at[pl.ds(r0, ROWS_TILE)],
                    part_hbm.at[cid, pl.ds(r0, ROWS_TILE)])


def _mm_body(p_ref, w_ref, b_ref, z_ref):
    agg = p_ref[0] + p_ref[1]
    z = jnp.dot(agg, w_ref[...], preferred_element_type=jnp.float32)
    z_ref[...] = jnp.maximum(z + b_ref[...], 0.0)


def _encode_mm(parts, W, b2d):
    blk = 632
    return pl.pallas_call(
        _mm_body,
        grid=(N_PAD // blk,),
        in_specs=[
            pl.BlockSpec((NC, blk, D_FEAT), lambda i: (0, i, 0)),
            pl.BlockSpec((D_FEAT, D_FEAT), lambda i: (0, 0)),
            pl.BlockSpec((1, D_FEAT), lambda i: (0, 0)),
        ],
        out_specs=pl.BlockSpec((blk, D_FEAT), lambda i: (i, 0)),
        out_shape=jax.ShapeDtypeStruct((N_PAD, D_FEAT), jnp.float32),
    )(parts, W, b2d)


@functools.partial(
    pl.kernel,
    out_type=jax.ShapeDtypeStruct((N_EDGES,), jnp.float32),
    mesh=_mesh,
    scratch_types=[
        pltpu.VMEM((CHUNK,), jnp.int32),
        pltpu.VMEM((CHUNK,), jnp.int32),
        pltpu.VMEM((CHUNK, D_FEAT), jnp.float32),
        pltpu.VMEM((CHUNK, D_FEAT), jnp.float32),
        pltpu.VMEM((CHUNK,), jnp.float32),
        pltpu.SemaphoreType.DMA,
        pltpu.SemaphoreType.DMA,
    ],
)
def _decode(z_hbm, src_hbm, dst_hbm, out_hbm,
            src_v, dst_v, srows_v, drows_v, out_v, sem_s, sem_d):
    cid = lax.axis_index("c")
    sid = lax.axis_index("s")
    wid = cid * NS + sid
    base_w = wid * E_W
    lane = lax.iota(jnp.int32, L)

    def body(i, carry):
        base = base_w + i * CHUNK
        pltpu.sync_copy(src_hbm.at[pl.ds(base, CHUNK)], src_v)
        pltpu.sync_copy(dst_hbm.at[pl.ds(base, CHUNK)], dst_v)
        cps = pltpu.async_copy(z_hbm.at[src_v], srows_v, sem_s)
        cpd = pltpu.async_copy(z_hbm.at[dst_v], drows_v, sem_d)
        cps.wait()
        cpd.wait()

        def group(g, carry):
            res = jnp.zeros((L,), jnp.float32)
            for j in range(L):
                acc = jnp.zeros((L,), jnp.float32)
                e = g * L + j
                for c in range(D_FEAT // L):
                    sv = srows_v[e, pl.ds(c * L, L)]
                    dv = drows_v[e, pl.ds(c * L, L)]
                    acc = acc + sv * dv
                for s in (8, 4, 2, 1):
                    acc = acc + _lane_shuffle(acc, jnp.bitwise_xor(lane, s))
                res = jnp.where(lane == j, acc, res)
            out_v[pl.ds(g * L, L)] = res
            return carry

        lax.fori_loop(0, CHUNK // L, group, 0)
        pltpu.sync_copy(out_v, out_hbm.at[pl.ds(base, CHUNK)])
        return carry

    lax.fori_loop(0, N_CHUNK, body, 0)


def kernel(x, edge_index, W, b):
    src = edge_index[0].astype(jnp.int32)
    dst = edge_index[1].astype(jnp.int32)
    zeros = jnp.zeros((N_PAD, D_FEAT), jnp.float32)
    parts = _segsum(x, src, dst, zeros)
    z = _encode_mm(parts, W, b.reshape(1, D_FEAT))
    return _decode(z, src, dst)


# speedup vs baseline: 7.3855x; 1.8773x over previous
import functools

import jax
import jax.numpy as jnp
from jax import lax
from jax.experimental import pallas as pl
from jax.experimental.pallas import tpu as pltpu
from jax.experimental.pallas import tpu_sc as plsc

N_NODES = 10000
D_FEAT = 128
N_EDGES = 320000

NC = 2
NS = 16
NW = NC * NS
L = 16

E_W = N_EDGES // NW
CHUNK = 80
N_CHUNK = E_W // CHUNK
N_PAIR = (N_CHUNK - 1) // 2
K16 = CHUNK // L
N_PAD = 10112
ROWS_TILE = N_PAD // NS

_mesh = plsc.VectorSubcoreMesh(core_axis_name="c", subcore_axis_name="s")

_GDN = lax.GatherDimensionNumbers(
    offset_dims=(), collapsed_slice_dims=(0,), start_index_map=(0,))


def _lane_shuffle(v, perm):
    return lax.gather(v, perm[:, None], _GDN, slice_sizes=(1,),
                      mode=lax.GatherScatterMode.PROMISE_IN_BOUNDS)


def _fill_small(small, big, c):
    for k in range(K16):
        small[pl.ds(k * L, L)] = big[pl.ds(c * CHUNK + k * L, L)]


@functools.partial(
    pl.kernel,
    out_type=jax.ShapeDtypeStruct((NC, N_PAD, D_FEAT), jnp.float32),
    mesh=_mesh,
    scratch_types=[
        pltpu.VMEM((E_W,), jnp.int32),
        pltpu.VMEM((E_W,), jnp.int32),
        pltpu.VMEM((CHUNK,), jnp.int32),
        pltpu.VMEM((CHUNK,), jnp.int32),
        pltpu.VMEM((CHUNK,), jnp.int32),
        pltpu.VMEM((CHUNK,), jnp.int32),
        pltpu.VMEM((CHUNK, D_FEAT), jnp.float32),
        pltpu.VMEM((CHUNK, D_FEAT), jnp.float32),
        pltpu.MemorySpace.VMEM_SHARED((N_PAD, D_FEAT), jnp.float32),
        pltpu.SemaphoreType.DMA,
        pltpu.SemaphoreType.DMA,
        pltpu.SemaphoreType.DMA,
        pltpu.SemaphoreType.DMA,
    ],
)
def _segsum(x_hbm, src_hbm, dst_hbm, zeros_hbm, part_hbm,
            src_big, dst_big, srcs0, srcs1, dsts0, dsts1, rows0, rows1,
            acc_sh, gsem0, gsem1, ssem0, ssem1):
    cid = lax.axis_index("c")
    sid = lax.axis_index("s")
    wid = cid * NS + sid
    r0 = sid * ROWS_TILE
    srcs = (srcs0, srcs1)
    dsts = (dsts0, dsts1)
    rows = (rows0, rows1)
    gsem = (gsem0, gsem1)
    ssem = (ssem0, ssem1)

    def gstart(b):
        pltpu.async_copy(x_hbm.at[srcs[b]], rows[b], gsem[b])

    def gwait(b):
        pltpu.make_async_copy(x_hbm.at[srcs[b]], rows[b], gsem[b]).wait()

    def sstart(b):
        pltpu.async_copy(rows[b], acc_sh.at[dsts[b]], ssem[b], add=True)

    def swait(b):
        pltpu.make_async_copy(rows[b], acc_sh.at[dsts[b]], ssem[b]).wait()

    pltpu.sync_copy(zeros_hbm.at[pl.ds(r0, ROWS_TILE)],
                    acc_sh.at[pl.ds(r0, ROWS_TILE)])
    base_w = wid * E_W
    pltpu.sync_copy(src_hbm.at[pl.ds(base_w, E_W)], src_big)
    pltpu.sync_copy(dst_hbm.at[pl.ds(base_w, E_W)], dst_big)
    _fill_small(srcs[0], src_big, 0)
    gstart(0)
    plsc.subcore_barrier()

    def pair(p, carry):
        for b in range(2):
            c = 2 * p + b
            gwait(b)
            if b == 0:
                @pl.when(p >= 1)
                def _():
                    swait(1)
            else:
                swait(0)
            _fill_small(srcs[1 - b], src_big, c + 1)
            gstart(1 - b)
            _fill_small(dsts[b], dst_big, c)
            sstart(b)
        return carry

    lax.fori_loop(0, N_PAIR, pair, 0)

    gwait(0)
    swait(1)
    _fill_small(dsts[0], dst_big, N_CHUNK - 1)
    sstart(0)
    swait(0)

    plsc.subcore_barrier()
    pltpu.sync_copy(acc_sh.at[pl.ds(r0, ROWS_TILE)],
                    part_hbm.at[cid, pl.ds(r0, ROWS_TILE)])


def _mm_body(p_ref, w_ref, b_ref, z_ref):
    agg = p_ref[0] + p_ref[1]
    z = jnp.dot(agg, w_ref[...], preferred_element_type=jnp.float32)
    z_ref[...] = jnp.maximum(z + b_ref[...], 0.0)


def _encode_mm(parts, W, b2d):
    blk = 632
    return pl.pallas_call(
        _mm_body,
        grid=(N_PAD // blk,),
        in_specs=[
            pl.BlockSpec((NC, blk, D_FEAT), lambda i: (0, i, 0)),
            pl.BlockSpec((D_FEAT, D_FEAT), lambda i: (0, 0)),
            pl.BlockSpec((1, D_FEAT), lambda i: (0, 0)),
        ],
        out_specs=pl.BlockSpec((blk, D_FEAT), lambda i: (i, 0)),
        out_shape=jax.ShapeDtypeStruct((N_PAD, D_FEAT), jnp.float32),
    )(parts, W, b2d)


@functools.partial(
    pl.kernel,
    out_type=jax.ShapeDtypeStruct((N_EDGES,), jnp.float32),
    mesh=_mesh,
    scratch_types=[
        pltpu.VMEM((E_W,), jnp.int32),
        pltpu.VMEM((E_W,), jnp.int32),
        pltpu.VMEM((CHUNK,), jnp.int32),
        pltpu.VMEM((CHUNK,), jnp.int32),
        pltpu.VMEM((CHUNK,), jnp.int32),
        pltpu.VMEM((CHUNK,), jnp.int32),
        pltpu.VMEM((CHUNK, D_FEAT), jnp.float32),
        pltpu.VMEM((CHUNK, D_FEAT), jnp.float32),
        pltpu.VMEM((CHUNK, D_FEAT), jnp.float32),
        pltpu.VMEM((CHUNK, D_FEAT), jnp.float32),
        pltpu.VMEM((CHUNK,), jnp.float32),
        pltpu.VMEM((CHUNK,), jnp.float32),
        pltpu.SemaphoreType.DMA,
        pltpu.SemaphoreType.DMA,
        pltpu.SemaphoreType.DMA,
        pltpu.SemaphoreType.DMA,
        pltpu.SemaphoreType.DMA,
        pltpu.SemaphoreType.DMA,
    ],
)
def _decode(z_hbm, src_hbm, dst_hbm, out_hbm,
            src_big, dst_big, srcs0, srcs1, dsts0, dsts1,
            srows0, srows1, drows0, drows1, outv0, outv1,
            gs0, gs1, gd0, gd1, os0, os1):
    cid = lax.axis_index("c")
    sid = lax.axis_index("s")
    wid = cid * NS + sid
    base_w = wid * E_W
    lane = lax.iota(jnp.int32, L)
    srcs = (srcs0, srcs1)
    dsts = (dsts0, dsts1)
    srows = (srows0, srows1)
    drows = (drows0, drows1)
    outv = (outv0, outv1)
    gs = (gs0, gs1)
    gd = (gd0, gd1)
    osem = (os0, os1)

    def gstart(b):
        pltpu.async_copy(z_hbm.at[srcs[b]], srows[b], gs[b])
        pltpu.async_copy(z_hbm.at[dsts[b]], drows[b], gd[b])

    def gwait(b):
        pltpu.make_async_copy(z_hbm.at[srcs[b]], srows[b], gs[b]).wait()
        pltpu.make_async_copy(z_hbm.at[dsts[b]], drows[b], gd[b]).wait()

    def ostart(b, c):
        pltpu.async_copy(outv[b], out_hbm.at[pl.ds(base_w + c * CHUNK, CHUNK)],
                         osem[b])

    def owait(b, c):
        pltpu.make_async_copy(
            outv[b], out_hbm.at[pl.ds(base_w + c * CHUNK, CHUNK)],
            osem[b]).wait()

    def compute(b):
        def group(g, carry):
            res = jnp.zeros((L,), jnp.float32)
            for j in range(L):
                e = g * L + j
                acc0 = jnp.zeros((L,), jnp.float32)
                acc1 = jnp.zeros((L,), jnp.float32)
                for cc in range(D_FEAT // L):
                    sv = srows[b][e, pl.ds(cc * L, L)]
                    dv = drows[b][e, pl.ds(cc * L, L)]
                    if cc % 2 == 0:
                        acc0 = acc0 + sv * dv
                    else:
                        acc1 = acc1 + sv * dv
                acc = acc0 + acc1
                for s in (8, 4, 2, 1):
                    acc = acc + _lane_shuffle(acc, jnp.bitwise_xor(lane, s))
                res = jnp.where(lane == j, acc, res)
            outv[b][pl.ds(g * L, L)] = res
            return carry

        lax.fori_loop(0, K16, group, 0)

    pltpu.sync_copy(src_hbm.at[pl.ds(base_w, E_W)], src_big)
    pltpu.sync_copy(dst_hbm.at[pl.ds(base_w, E_W)], dst_big)
    _fill_small(srcs[0], src_big, 0)
    _fill_small(dsts[0], dst_big, 0)
    gstart(0)

    def pair(p, carry):
        for b in range(2):
            c = 2 * p + b
            gwait(b)
            _fill_small(srcs[1 - b], src_big, c + 1)
            _fill_small(dsts[1 - b], dst_big, c + 1)
            gstart(1 - b)

            @pl.when(p >= 1)
            def _():
                owait(b, c - 2)

            compute(b)
            ostart(b, c)
        return carry

    lax.fori_loop(0, N_PAIR, pair, 0)

    cl = N_CHUNK - 1
    gwait(0)
    owait(0, cl - 2)
    compute(0)
    ostart(0, cl)
    owait(0, cl)
    owait(1, cl - 1)


def kernel(x, edge_index, W, b):
    src = edge_index[0].astype(jnp.int32)
    dst = edge_index[1].astype(jnp.int32)
    zeros = jnp.zeros((N_PAD, D_FEAT), jnp.float32)
    parts = _segsum(x, src, dst, zeros)
    z = _encode_mm(parts, W, b.reshape(1, D_FEAT))
    return _decode(z, src, dst)


# speedup vs baseline: 8.7191x; 1.1806x over previous
import functools

import jax
import jax.numpy as jnp
from jax import lax
from jax.experimental import pallas as pl
from jax.experimental.pallas import tpu as pltpu
from jax.experimental.pallas import tpu_sc as plsc

N_NODES = 10000
D_FEAT = 128
N_EDGES = 320000

NC = 2
NS = 16
NW = NC * NS
L = 16

E_W = N_EDGES // NW
CHUNK = 80
N_CHUNK = E_W // CHUNK
NBUF = 5
N_GRP = N_CHUNK // NBUF
AHEAD = 3
K16 = CHUNK // L
CHUNK_A = 40
N_CHUNK_A = E_W // CHUNK_A
N_GRP_A = N_CHUNK_A // NBUF
N_PAD = 10112
ROWS_TILE = N_PAD // NS

_mesh = plsc.VectorSubcoreMesh(core_axis_name="c", subcore_axis_name="s")

_GDN = lax.GatherDimensionNumbers(
    offset_dims=(), collapsed_slice_dims=(0,), start_index_map=(0,))


def _lane_shuffle(v, perm):
    return lax.gather(v, perm[:, None], _GDN, slice_sizes=(1,),
                      mode=lax.GatherScatterMode.PROMISE_IN_BOUNDS)


@functools.partial(
    pl.kernel,
    out_type=jax.ShapeDtypeStruct((NC, N_PAD, D_FEAT), jnp.float32),
    mesh=_mesh,
    scratch_types=[
        pltpu.VMEM((E_W,), jnp.int32),
        [pltpu.VMEM((CHUNK_A,), jnp.int32)] * NBUF,
        [pltpu.VMEM((CHUNK_A, D_FEAT), jnp.float32)] * NBUF,
        pltpu.MemorySpace.VMEM_SHARED((N_PAD, D_FEAT), jnp.float32),
        [pltpu.SemaphoreType.DMA] * NBUF,
        [pltpu.SemaphoreType.DMA] * NBUF,
        [pltpu.SemaphoreType.DMA] * NBUF,
    ],
)
def _segsum(x_hbm, src_hbm, dst_hbm, zeros_hbm, part_hbm,
            src_big, dsts, rows, acc_sh, gsem, ssem, isem):
    cid = lax.axis_index("c")
    sid = lax.axis_index("s")
    wid = cid * NS + sid
    r0 = sid * ROWS_TILE
    base_w = wid * E_W

    def gstart(b, c):
        pltpu.async_copy(x_hbm.at[src_big.at[pl.ds(c * CHUNK_A, CHUNK_A)]],
                         rows[b], gsem[b])

    def gwait(b, c):
        pltpu.make_async_copy(
            x_hbm.at[src_big.at[pl.ds(c * CHUNK_A, CHUNK_A)]],
            rows[b], gsem[b]).wait()

    def istart(b, c):
        pltpu.async_copy(dst_hbm.at[pl.ds(base_w + c * CHUNK_A, CHUNK_A)],
                         dsts[b], isem[b])

    def iwait(b, c):
        pltpu.make_async_copy(
            dst_hbm.at[pl.ds(base_w + c * CHUNK_A, CHUNK_A)],
            dsts[b], isem[b]).wait()

    def sstart(b):
        pltpu.async_copy(rows[b], acc_sh.at[dsts[b]], ssem[b], add=True)

    def swait(b):
        pltpu.make_async_copy(rows[b], acc_sh.at[dsts[b]], ssem[b]).wait()

    pltpu.sync_copy(zeros_hbm.at[pl.ds(r0, ROWS_TILE)],
                    acc_sh.at[pl.ds(r0, ROWS_TILE)])
    pltpu.sync_copy(src_hbm.at[pl.ds(base_w, E_W)], src_big)
    for b in range(AHEAD):
        istart(b, b)
        gstart(b, b)
    plsc.subcore_barrier()

    def grp(g, carry):
        for bb in range(NBUF):
            c = NBUF * g + bb
            gwait(bb, c)
            nb = (bb + AHEAD) % NBUF

            def ahead(bb=bb, nb=nb, c=c):
                if bb < NBUF - AHEAD:
                    pl.when(g >= 1)(lambda: swait(nb))
                else:
                    swait(nb)
                gstart(nb, c + AHEAD)
                istart(nb, c + AHEAD)

            if bb < NBUF - AHEAD:
                ahead()
            else:
                pl.when(g < N_GRP_A - 1)(ahead)
            iwait(bb, c)
            sstart(bb)
        return carry

    lax.fori_loop(0, N_GRP_A, grp, 0)

    for b in range(NBUF):
        swait(b)
    plsc.subcore_barrier()
    pltpu.sync_copy(acc_sh.at[pl.ds(r0, ROWS_TILE)],
                    part_hbm.at[cid, pl.ds(r0, ROWS_TILE)])


def _mm_body(p_ref, w_ref, b_ref, z_ref):
    agg = p_ref[0] + p_ref[1]
    z = jnp.dot(agg, w_ref[...], preferred_element_type=jnp.float32)
    z_ref[...] = jnp.maximum(z + b_ref[...], 0.0)


def _encode_mm(parts, W, b2d):
    blk = 632
    return pl.pallas_call(
        _mm_body,
        grid=(N_PAD // blk,),
        in_specs=[
            pl.BlockSpec((NC, blk, D_FEAT), lambda i: (0, i, 0)),
            pl.BlockSpec((D_FEAT, D_FEAT), lambda i: (0, 0)),
            pl.BlockSpec((1, D_FEAT), lambda i: (0, 0)),
        ],
        out_specs=pl.BlockSpec((blk, D_FEAT), lambda i: (i, 0)),
        out_shape=jax.ShapeDtypeStruct((N_PAD, D_FEAT), jnp.float32),
    )(parts, W, b2d)


@functools.partial(
    pl.kernel,
    out_type=jax.ShapeDtypeStruct((N_EDGES,), jnp.float32),
    mesh=_mesh,
    scratch_types=[
        [pltpu.VMEM((CHUNK,), jnp.int32)] * NBUF,
        [pltpu.VMEM((CHUNK,), jnp.int32)] * NBUF,
        [pltpu.VMEM((CHUNK, D_FEAT), jnp.float32)] * NBUF,
        [pltpu.VMEM((CHUNK, D_FEAT), jnp.float32)] * NBUF,
        [pltpu.VMEM((CHUNK,), jnp.float32)] * NBUF,
        [pltpu.SemaphoreType.DMA] * NBUF,
        [pltpu.SemaphoreType.DMA] * NBUF,
        [pltpu.SemaphoreType.DMA] * NBUF,
        [pltpu.SemaphoreType.DMA] * NBUF,
        [pltpu.SemaphoreType.DMA] * NBUF,
    ],
)
def _decode(z_hbm, src_hbm, dst_hbm, out_hbm,
            srcs, dsts, srows, drows, outv,
            gs, gd, osem, iss, ids):
    cid = lax.axis_index("c")
    sid = lax.axis_index("s")
    wid = cid * NS + sid
    base_w = wid * E_W
    lane = lax.iota(jnp.int32, L)

    def istart(b, c):
        pltpu.async_copy(src_hbm.at[pl.ds(base_w + c * CHUNK, CHUNK)],
                         srcs[b], iss[b])
        pltpu.async_copy(dst_hbm.at[pl.ds(base_w + c * CHUNK, CHUNK)],
                         dsts[b], ids[b])

    def iwait(b, c):
        pltpu.make_async_copy(
            src_hbm.at[pl.ds(base_w + c * CHUNK, CHUNK)],
            srcs[b], iss[b]).wait()
        pltpu.make_async_copy(
            dst_hbm.at[pl.ds(base_w + c * CHUNK, CHUNK)],
            dsts[b], ids[b]).wait()

    def gstart(b):
        pltpu.async_copy(z_hbm.at[srcs[b]], srows[b], gs[b])
        pltpu.async_copy(z_hbm.at[dsts[b]], drows[b], gd[b])

    def gwait(b):
        pltpu.make_async_copy(z_hbm.at[srcs[b]], srows[b], gs[b]).wait()
        pltpu.make_async_copy(z_hbm.at[dsts[b]], drows[b], gd[b]).wait()

    def ostart(b, c):
        pltpu.async_copy(outv[b], out_hbm.at[pl.ds(base_w + c * CHUNK, CHUNK)],
                         osem[b])

    def owait(b, c):
        pltpu.make_async_copy(
            outv[b], out_hbm.at[pl.ds(base_w + c * CHUNK, CHUNK)],
            osem[b]).wait()

    def compute(b):
        def group(g2, carry):
            stack = []
            for j in range(L):
                e = g2 * L + j
                a = jnp.zeros((L,), jnp.float32)
                for cc in range(D_FEAT // L):
                    sv = srows[b][e, pl.ds(cc * L, L)]
                    dv = drows[b][e, pl.ds(cc * L, L)]
                    a = a + sv * dv
                rank, cur = 0, a
                while stack and stack[-1][0] == rank:
                    _, u = stack.pop()
                    d = 1 << rank
                    m = (lane & d) == 0
                    su = _lane_shuffle(u, jnp.bitwise_xor(lane, d))
                    sc = _lane_shuffle(cur, jnp.bitwise_xor(lane, d))
                    cur = jnp.where(m, u, cur) + jnp.where(m, su, sc)
                    rank += 1
                stack.append((rank, cur))
            outv[b][pl.ds(g2 * L, L)] = stack[0][1]
            return carry

        lax.fori_loop(0, K16, group, 0)

    for b in range(AHEAD + 1):
        istart(b, b)
    for b in range(AHEAD):
        iwait(b, b)
        gstart(b)

    def grp(g, carry):
        for bb in range(NBUF):
            c = NBUF * g + bb
            gwait(bb)
            nb = (bb + AHEAD) % NBUF
            nb2 = (bb + AHEAD + 1) % NBUF

            def ahead(bb=bb, nb=nb, c=c):
                iwait(nb, c + AHEAD)
                gstart(nb)

            def ahead2(nb2=nb2, c=c):
                istart(nb2, c + AHEAD + 1)

            if bb < NBUF - AHEAD:
                ahead()
            else:
                pl.when(g < N_GRP - 1)(ahead)
            if bb < NBUF - AHEAD - 1:
                ahead2()
            else:
                pl.when(g < N_GRP - 1)(ahead2)
            pl.when(g >= 1)(lambda bb=bb, c=c: owait(bb, c - NBUF))
            compute(bb)
            ostart(bb, c)
        return carry

    lax.fori_loop(0, N_GRP, grp, 0)

    for b in range(NBUF):
        owait(b, (N_GRP - 1) * NBUF + b)


def kernel(x, edge_index, W, b):
    src = edge_index[0].astype(jnp.int32)
    dst = edge_index[1].astype(jnp.int32)
    zeros = jnp.zeros((N_PAD, D_FEAT), jnp.float32)
    parts = _segsum(x, src, dst, zeros)
    z = _encode_mm(parts, W, b.reshape(1, D_FEAT))
    return _decode(z, src, dst)
